# baseline (speedup 1.0000x reference)
import jax
import jax.numpy as jnp
from jax.experimental import pallas as pl
from jax.experimental.pallas import tpu as pltpu

_TT = 512


def _top2_combine(logits):
    E = logits.shape[-1]
    eids = jax.lax.broadcasted_iota(jnp.int32, logits.shape, 1)
    m1 = jnp.max(logits, axis=1, keepdims=True)
    i1 = jnp.min(jnp.where(logits == m1, eids, E), axis=1, keepdims=True)
    mask1 = eids == i1
    masked = jnp.where(mask1, -jnp.inf, logits)
    m2 = jnp.max(masked, axis=1, keepdims=True)
    i2 = jnp.min(jnp.where(masked == m2, eids, E), axis=1, keepdims=True)
    mask2 = eids == i2
    e2 = jnp.exp(m2 - m1)
    w1 = 1.0 / (1.0 + e2)
    w2 = e2 / (1.0 + e2)
    return w1 * mask1.astype(logits.dtype) + w2 * mask2.astype(logits.dtype)


def _moe_kernel(x_ref, wg_ref, we_hbm, be_ref, out_ref, we_v, sems):
    i = pl.program_id(0)
    E = be_ref.shape[0]

    @pl.when(i == 0)
    def _start_dmas():
        for e in range(E):
            pltpu.make_async_copy(we_hbm.at[e], we_v.at[e], sems.at[e]).start()

    x = x_ref[...]
    logits = jnp.dot(x, wg_ref[...], preferred_element_type=jnp.float32)
    combine = _top2_combine(logits)
    xb = x.astype(jnp.bfloat16)
    acc = jnp.dot(combine, be_ref[...], preferred_element_type=jnp.float32)
    for e in range(E):
        @pl.when(i == 0)
        def _wait_dma(e=e):
            pltpu.make_async_copy(we_hbm.at[e], we_v.at[e], sems.at[e]).wait()

        ye = jnp.dot(xb, we_v[e].astype(jnp.bfloat16),
                     preferred_element_type=jnp.float32)
        acc = acc + combine[:, e:e + 1] * ye
    out_ref[...] = acc


def kernel(inputs, Wg, We, be):
    D = inputs.shape[-1]
    E = We.shape[0]
    xf = inputs.reshape(-1, D)
    T = xf.shape[0]
    out = pl.pallas_call(
        _moe_kernel,
        grid=(T // _TT,),
        in_specs=[
            pl.BlockSpec((_TT, D), lambda i: (i, 0)),
            pl.BlockSpec(Wg.shape, lambda i: (0, 0)),
            pl.BlockSpec(memory_space=pl.ANY),
            pl.BlockSpec(be.shape, lambda i: (0, 0)),
        ],
        out_specs=pl.BlockSpec((_TT, D), lambda i: (i, 0)),
        out_shape=jax.ShapeDtypeStruct((T, D), inputs.dtype),
        scratch_shapes=[
            pltpu.VMEM((E, D, D), jnp.float32),
            pltpu.SemaphoreType.DMA((E,)),
        ],
        compiler_params=pltpu.CompilerParams(
            dimension_semantics=("arbitrary",)),
    )(xf, Wg, We, be)
    return out.reshape(inputs.shape)


# speedup vs baseline: 1.1513x; 1.1513x over previous
import jax
import jax.numpy as jnp
from jax.experimental import pallas as pl
from jax.experimental.pallas import tpu as pltpu

_TT = 1024


def _top2_combine(logits):
    E = logits.shape[-1]
    eids = jax.lax.broadcasted_iota(jnp.int32, logits.shape, 1)
    m1 = jnp.max(logits, axis=1, keepdims=True)
    i1 = jnp.min(jnp.where(logits == m1, eids, E), axis=1, keepdims=True)
    mask1 = eids == i1
    masked = jnp.where(mask1, -jnp.inf, logits)
    m2 = jnp.max(masked, axis=1, keepdims=True)
    i2 = jnp.min(jnp.where(masked == m2, eids, E), axis=1, keepdims=True)
    mask2 = eids == i2
    e2 = jnp.exp(m2 - m1)
    w1 = 1.0 / (1.0 + e2)
    w2 = e2 / (1.0 + e2)
    return w1 * mask1.astype(logits.dtype) + w2 * mask2.astype(logits.dtype)


def _moe_kernel(x_ref, wg_ref, we_ref, be_ref, out_ref):
    x = x_ref[...]
    logits = jnp.dot(x, wg_ref[...], preferred_element_type=jnp.float32)
    combine = _top2_combine(logits)
    E = logits.shape[-1]
    xb = x.astype(jnp.bfloat16)
    acc = jnp.dot(combine, be_ref[...], preferred_element_type=jnp.float32)
    for e in range(E):
        ye = jnp.dot(xb, we_ref[e].astype(jnp.bfloat16),
                     preferred_element_type=jnp.float32)
        acc = acc + combine[:, e:e + 1] * ye
    out_ref[...] = acc


def kernel(inputs, Wg, We, be):
    D = inputs.shape[-1]
    xf = inputs.reshape(-1, D)
    T = xf.shape[0]
    out = pl.pallas_call(
        _moe_kernel,
        grid=(T // _TT,),
        in_specs=[
            pl.BlockSpec((_TT, D), lambda i: (i, 0)),
            pl.BlockSpec(Wg.shape, lambda i: (0, 0)),
            pl.BlockSpec(We.shape, lambda i: (0, 0, 0)),
            pl.BlockSpec(be.shape, lambda i: (0, 0)),
        ],
        out_specs=pl.BlockSpec((_TT, D), lambda i: (i, 0)),
        out_shape=jax.ShapeDtypeStruct((T, D), inputs.dtype),
    )(xf, Wg, We, be)
    return out.reshape(inputs.shape)


# speedup vs baseline: 1.1517x; 1.0003x over previous
import jax
import jax.numpy as jnp
from jax.experimental import pallas as pl
_TT = 1024


def _top2_combine(logits):
    E = logits.shape[-1]
    eids = jax.lax.broadcasted_iota(jnp.int32, logits.shape, 1)
    m1 = jnp.max(logits, axis=1, keepdims=True)
    i1 = jnp.min(jnp.where(logits == m1, eids, E), axis=1, keepdims=True)
    mask1 = eids == i1
    masked = jnp.where(mask1, -jnp.inf, logits)
    m2 = jnp.max(masked, axis=1, keepdims=True)
    i2 = jnp.min(jnp.where(masked == m2, eids, E), axis=1, keepdims=True)
    mask2 = eids == i2
    e2 = jnp.exp(m2 - m1)
    w1 = 1.0 / (1.0 + e2)
    w2 = e2 / (1.0 + e2)
    return w1 * mask1.astype(logits.dtype) + w2 * mask2.astype(logits.dtype)


def _moe_kernel(x_ref, wg_ref, we_ref, be_ref, out_ref):
    x = x_ref[...]
    logits = jnp.dot(x, wg_ref[...], preferred_element_type=jnp.float32)
    combine = _top2_combine(logits)
    E = logits.shape[-1]
    xb = x.astype(jnp.bfloat16)
    acc = jnp.dot(combine, be_ref[...], preferred_element_type=jnp.float32)
    for e in range(E):
        ye = jnp.dot(xb, we_ref[e].astype(jnp.bfloat16),
                     preferred_element_type=jnp.float32)
        acc = acc + combine[:, e:e + 1] * ye
    out_ref[...] = acc


def kernel(inputs, Wg, We, be):
    D = inputs.shape[-1]
    xf = inputs.reshape(-1, D)
    T = xf.shape[0]
    out = pl.pallas_call(
        _moe_kernel,
        grid=(T // _TT,),
        in_specs=[
            pl.BlockSpec((_TT, D), lambda i: (i, 0)),
            pl.BlockSpec(Wg.shape, lambda i: (0, 0)),
            pl.BlockSpec(We.shape, lambda i: (0, 0, 0)),
            pl.BlockSpec(be.shape, lambda i: (0, 0)),
        ],
        out_specs=pl.BlockSpec((_TT, D), lambda i: (i, 0)),
        out_shape=jax.ShapeDtypeStruct((T, D), inputs.dtype),
    )(xf, Wg, We, be)
    return out.reshape(inputs.shape)
